# baseline (speedup 1.0000x reference)
import functools

import jax
import jax.numpy as jnp
from jax import lax
from jax.experimental import pallas as pl
from jax.experimental.pallas import tpu as pltpu
from jax.experimental.pallas import tpu_sc as plsc

H = 1024
EPS = 1e-12
L = 16
NC = 2
NS = 16
NW = NC * NS
HV = H // L
CH = 16
NB = 3


def _rsqrt_vec(x):
    i = plsc.bitcast(x, jnp.int32)
    i = jnp.full((L,), 0x5F3759DF, jnp.int32) - lax.shift_right_arithmetic(i, 1)
    y = plsc.bitcast(i, jnp.float32)
    for _ in range(3):
        y = y * (1.5 - 0.5 * x * y * y)
    return y


def _sc_embed_ln(ids_flat, token_table, pos_table, gamma, beta, *, nt, s_len):
    rpw = nt // NW
    nch = rpw // CH

    mesh = plsc.VectorSubcoreMesh(
        core_axis_name="c", subcore_axis_name="s",
        num_cores=NC, num_subcores=NS)

    @functools.partial(
        pl.kernel,
        out_type=jax.ShapeDtypeStruct((nt, H), jnp.float32),
        mesh=mesh,
        scratch_types=[
            pltpu.VMEM((rpw,), jnp.int32),
            pltpu.VMEM((H,), jnp.float32),
            pltpu.VMEM((H,), jnp.float32),
            pltpu.VMEM((NB, CH, H), jnp.float32),
            pltpu.VMEM((NB, CH, H), jnp.float32),
            pltpu.SemaphoreType.DMA((NB,)),
            pltpu.SemaphoreType.DMA((NB,)),
        ],
        compiler_params=pltpu.CompilerParams(needs_layout_passes=False),
    )
    def k(ids_hbm, tok_hbm, pos_hbm, gamma_hbm, beta_hbm, out_hbm,
          idx_v, gamma_v, beta_v, tok_b, pos_b, gsem, wsem):
        wid = lax.axis_index("s") * NC + lax.axis_index("c")
        base = wid * rpw
        pos0 = lax.rem(base, s_len)

        pltpu.sync_copy(ids_hbm.at[pl.ds(base, rpw)], idx_v)
        pltpu.sync_copy(gamma_hbm, gamma_v)
        pltpu.sync_copy(beta_hbm, beta_v)

        def start_chunk(c, b):
            pltpu.async_copy(
                tok_hbm.at[idx_v.at[pl.ds(c * CH, CH)]], tok_b.at[b],
                gsem.at[b])
            pltpu.async_copy(
                pos_hbm.at[pl.ds(pos0 + c * CH, CH)], pos_b.at[b],
                gsem.at[b])

        def wait_chunk(b):
            pltpu.make_async_copy(
                tok_hbm.at[pl.ds(0, CH)], tok_b.at[b], gsem.at[b]).wait()
            pltpu.make_async_copy(
                pos_hbm.at[pl.ds(0, CH)], pos_b.at[b], gsem.at[b]).wait()

        def wait_writeout(b):
            pltpu.make_async_copy(
                tok_b.at[b], out_hbm.at[pl.ds(0, CH)], wsem.at[b]).wait()

        start_chunk(0, 0)

        @pl.loop(0, nch)
        def chunk_loop(c):
            b = lax.rem(c, NB)
            bp = lax.rem(c + 1, NB)

            @pl.when(c + 1 < nch)
            def _prefetch():
                @pl.when(c >= NB - 1)
                def _reuse_guard():
                    wait_writeout(bp)
                start_chunk(c + 1, bp)

            wait_chunk(b)

            bidx = jnp.full((L,), b, jnp.int32)
            rows = lax.iota(jnp.int32, L)

            def col_body(j, carry):
                s1, s2 = carry
                jv = jnp.full((L,), 1, jnp.int32) * j
                t = plsc.load_gather(tok_b, [bidx, rows, jv])
                p = plsc.load_gather(pos_b, [bidx, rows, jv])
                v = t + p
                plsc.store_scatter(tok_b, [bidx, rows, jv], v)
                return s1 + v, s2 + v * v

            zero = jnp.zeros((L,), jnp.float32)
            s1, s2 = lax.fori_loop(0, H, col_body, (zero, zero), unroll=4)

            mean_v = s1 * (1.0 / H)
            var_v = s2 * (1.0 / H) - mean_v * mean_v
            rstd_v = _rsqrt_vec(var_v + EPS)
            d_v = -mean_v * rstd_v

            a_sp = [jnp.full((L,), rstd_v[r]) for r in range(CH)]
            d_sp = [jnp.full((L,), d_v[r]) for r in range(CH)]

            def grp_body(j, _):
                g = gamma_v[pl.ds(j * L, L)]
                bb = beta_v[pl.ds(j * L, L)]
                for r in range(CH):
                    v = tok_b[b, r, pl.ds(j * L, L)]
                    t = v * a_sp[r] + d_sp[r]
                    tok_b[b, r, pl.ds(j * L, L)] = t * g + bb
                return 0

            lax.fori_loop(0, HV, grp_body, 0)

            pltpu.async_copy(
                tok_b.at[b], out_hbm.at[pl.ds(base + c * CH, CH)], wsem.at[b])

        for i in range(min(NB, nch)):
            wait_writeout((nch - 1 - i) % NB)

    return k(ids_flat, token_table, pos_table, gamma, beta)


def kernel(input_ids, token_table, pos_table, gamma, beta):
    b, s = input_ids.shape
    nt = b * s
    ids_flat = input_ids.reshape(nt).astype(jnp.int32)
    out = _sc_embed_ln(ids_flat, token_table, pos_table, gamma, beta,
                       nt=nt, s_len=s)
    return out.reshape(b, s, H)


# speedup vs baseline: 5.3379x; 5.3379x over previous
import functools

import jax
import jax.numpy as jnp
from jax import lax
from jax.experimental import pallas as pl
from jax.experimental.pallas import tpu as pltpu
from jax.experimental.pallas import tpu_sc as plsc

H = 1024
EPS = 1e-12
L = 16
NC = 2
NS = 16
NW = NC * NS
HV = H // L
CH = 16
NB = 3


def _rsqrt_vec(x):
    i = plsc.bitcast(x, jnp.int32)
    i = jnp.full((L,), 0x5F3759DF, jnp.int32) - lax.shift_right_arithmetic(i, 1)
    y = plsc.bitcast(i, jnp.float32)
    for _ in range(3):
        y = y * (1.5 - 0.5 * x * y * y)
    return y


def _sc_embed_ln(ids_flat, token_table, pos_table, gamma, beta, *, nt, s_len):
    rpw = nt // NW
    nch = rpw // CH

    mesh = plsc.VectorSubcoreMesh(
        core_axis_name="c", subcore_axis_name="s",
        num_cores=NC, num_subcores=NS)

    @functools.partial(
        pl.kernel,
        out_type=jax.ShapeDtypeStruct((nt, H), jnp.float32),
        mesh=mesh,
        scratch_types=[
            pltpu.VMEM((rpw,), jnp.int32),
            pltpu.VMEM((H,), jnp.float32),
            pltpu.VMEM((H,), jnp.float32),
            pltpu.VMEM((NB, CH, H), jnp.float32),
            pltpu.VMEM((NB, CH, H), jnp.float32),
            pltpu.VMEM((L, 33), jnp.float32),
            pltpu.SemaphoreType.DMA((NB,)),
            pltpu.SemaphoreType.DMA((NB,)),
        ],
        compiler_params=pltpu.CompilerParams(needs_layout_passes=False),
    )
    def k(ids_hbm, tok_hbm, pos_hbm, gamma_hbm, beta_hbm, out_hbm,
          idx_v, gamma_v, beta_v, tok_b, pos_b, red_v, gsem, wsem):
        wid = lax.axis_index("s") * NC + lax.axis_index("c")
        base = wid * rpw
        pos0 = lax.rem(base, s_len)

        pltpu.sync_copy(ids_hbm.at[pl.ds(base, rpw)], idx_v)
        pltpu.sync_copy(gamma_hbm, gamma_v)
        pltpu.sync_copy(beta_hbm, beta_v)

        def start_chunk(c, b):
            pltpu.async_copy(
                tok_hbm.at[idx_v.at[pl.ds(c * CH, CH)]], tok_b.at[b],
                gsem.at[b])
            pltpu.async_copy(
                pos_hbm.at[pl.ds(pos0 + c * CH, CH)], pos_b.at[b],
                gsem.at[b])

        def wait_chunk(b):
            pltpu.make_async_copy(
                tok_hbm.at[pl.ds(0, CH)], tok_b.at[b], gsem.at[b]).wait()
            pltpu.make_async_copy(
                pos_hbm.at[pl.ds(0, CH)], pos_b.at[b], gsem.at[b]).wait()

        def wait_writeout(b):
            pltpu.make_async_copy(
                tok_b.at[b], out_hbm.at[pl.ds(0, CH)], wsem.at[b]).wait()

        start_chunk(0, 0)

        @pl.loop(0, nch)
        def chunk_loop(c):
            b = lax.rem(c, NB)
            bp = lax.rem(c + 1, NB)

            @pl.when(c + 1 < nch)
            def _prefetch():
                @pl.when(c >= NB - 1)
                def _reuse_guard():
                    wait_writeout(bp)
                start_chunk(c + 1, bp)

            wait_chunk(b)

            rows = lax.iota(jnp.int32, L)
            zero = jnp.zeros((L,), jnp.float32)
            for r in range(CH):
                def acc_body(j, carry, r=r):
                    s1, s2 = carry
                    v = (tok_b[b, r, pl.ds(j * L, L)]
                         + pos_b[b, r, pl.ds(j * L, L)])
                    tok_b[b, r, pl.ds(j * L, L)] = v
                    return s1 + v, s2 + v * v

                s1, s2 = lax.fori_loop(0, HV, acc_body, (zero, zero),
                                       unroll=4)
                rv = jnp.full((L,), r, jnp.int32)
                plsc.store_scatter(red_v, [rows, rv], s1)
                plsc.store_scatter(red_v, [rows, rv + L], s2)

            m1 = red_v[0, pl.ds(0, L)]
            m2 = red_v[0, pl.ds(L, L)]
            for i in range(1, L):
                m1 = m1 + red_v[i, pl.ds(0, L)]
                m2 = m2 + red_v[i, pl.ds(L, L)]

            mean_v = m1 * (1.0 / H)
            var_v = m2 * (1.0 / H) - mean_v * mean_v
            rstd_v = _rsqrt_vec(var_v + EPS)
            d_v = -mean_v * rstd_v

            a_sp = [jnp.full((L,), rstd_v[r]) for r in range(CH)]
            d_sp = [jnp.full((L,), d_v[r]) for r in range(CH)]

            def grp_body(j, _):
                g = gamma_v[pl.ds(j * L, L)]
                bb = beta_v[pl.ds(j * L, L)]
                for r in range(CH):
                    v = tok_b[b, r, pl.ds(j * L, L)]
                    t = v * a_sp[r] + d_sp[r]
                    tok_b[b, r, pl.ds(j * L, L)] = t * g + bb
                return 0

            lax.fori_loop(0, HV, grp_body, 0)

            pltpu.async_copy(
                tok_b.at[b], out_hbm.at[pl.ds(base + c * CH, CH)], wsem.at[b])

        for i in range(min(NB, nch)):
            wait_writeout((nch - 1 - i) % NB)

    return k(ids_flat, token_table, pos_table, gamma, beta)


def kernel(input_ids, token_table, pos_table, gamma, beta):
    b, s = input_ids.shape
    nt = b * s
    ids_flat = input_ids.reshape(nt).astype(jnp.int32)
    out = _sc_embed_ln(ids_flat, token_table, pos_table, gamma, beta,
                       nt=nt, s_len=s)
    return out.reshape(b, s, H)
